# Initial kernel scaffold; baseline (speedup 1.0000x reference)
import jax
import jax.numpy as jnp
from jax.experimental import pallas as pl


def kernel(x, edge_index, W, b):
    raise NotImplementedError("write your pallas kernel here")


# speedup vs baseline: 3.0961x; 3.0961x over previous
import functools

import jax
import jax.numpy as jnp
from jax import lax
from jax.experimental import pallas as pl
from jax.experimental.pallas import tpu as pltpu
from jax.experimental.pallas import tpu_sc as plsc

NC = 2
NS = 16
BLK = 128
DW = 128


def _deg_kernel_body(n_pad, bpw, dst_hbm, zeros_hbm, ones_hbm, out_hbm,
                     dstv, onesv, acc):
    c = lax.axis_index("c")
    s = lax.axis_index("s")
    rps = n_pad // NS
    pltpu.sync_copy(zeros_hbm, acc.at[pl.ds(s * rps, rps)])
    pltpu.sync_copy(ones_hbm, onesv)
    plsc.subcore_barrier()

    w = s * NC + c
    pltpu.sync_copy(dst_hbm.at[pl.ds(w * bpw, bpw)], dstv)

    def body(blk, _):
        pltpu.sync_copy(onesv, acc.at[dstv.at[blk]], add=True)
        return 0

    lax.fori_loop(0, bpw, body, 0)
    plsc.subcore_barrier()

    for core_id in range(NC):
        @pl.when(c == core_id)
        def _():
            pltpu.sync_copy(acc.at[pl.ds(s * rps, rps)],
                            out_hbm.at[core_id, pl.ds(s * rps, rps)])


def _hop_kernel_body(n_pad, d, bpw, g_hbm, src_hbm, dst_hbm, zeros_hbm,
                     out_hbm, srcv, dstv, rows, acc, sem):
    c = lax.axis_index("c")
    s = lax.axis_index("s")
    rps = n_pad // NS
    pltpu.sync_copy(zeros_hbm, acc.at[pl.ds(s * rps, rps)])
    plsc.subcore_barrier()

    w = s * NC + c
    pltpu.sync_copy(src_hbm.at[pl.ds(w * bpw, bpw)], srcv)
    pltpu.sync_copy(dst_hbm.at[pl.ds(w * bpw, bpw)], dstv)

    def body(blk, _):
        pltpu.async_copy(g_hbm.at[srcv.at[blk]], rows, sem).wait()
        pltpu.sync_copy(rows, acc.at[dstv.at[blk]], add=True)
        return 0

    lax.fori_loop(0, bpw, body, 0)
    plsc.subcore_barrier()

    for core_id in range(NC):
        @pl.when(c == core_id)
        def _():
            pltpu.sync_copy(acc.at[pl.ds(s * rps, rps)],
                            out_hbm.at[core_id, pl.ds(s * rps, rps)])


def _prep_body(parts_ref, x_ref, g0_ref, dinv_ref, dinvv_ref):
    deg = jnp.maximum(parts_ref[0, :, 0:1] + parts_ref[1, :, 0:1], 1.0)
    dinv = lax.rsqrt(deg)
    g0_ref[...] = x_ref[...] * dinv
    dinv_ref[...] = dinv
    dinvv_ref[...] = 1.0 / deg


def _comb_body(p_ref, s_ref, o_ref):
    o_ref[...] = (p_ref[0] + p_ref[1]) * s_ref[...]


def _final_body(n, p_ref, dinv_ref, w_ref, b_ref, o_ref):
    h = (p_ref[0, :n] + p_ref[1, :n]) * dinv_ref[:n]
    o_ref[...] = lax.dot_general(
        h, w_ref[...], (((1,), (1,)), ((), ())),
        preferred_element_type=jnp.float32,
        precision=lax.Precision.HIGHEST) + b_ref[...]


def kernel(x, edge_index, W, b):
    n, d = x.shape
    e = edge_index.shape[1]
    n_pad = ((n + 1279) // 1280) * 1280
    e_tot = e + n
    quant = NC * NS * BLK * 8
    e_pad = ((e_tot + quant - 1) // quant) * quant
    bpw = e_pad // (NC * NS * BLK)
    rps = n_pad // NS

    src = edge_index[0]
    dst = edge_index[1]
    loop = jnp.arange(n, dtype=jnp.int32)
    padv = jnp.full((e_pad - e_tot,), n, dtype=jnp.int32)
    src2d = jnp.concatenate([src, loop, padv]).reshape(e_pad // BLK, BLK)
    dst2d = jnp.concatenate([dst, loop, padv]).reshape(e_pad // BLK, BLK)
    x_pad = jnp.pad(x, ((0, n_pad - n), (0, 0)))
    zeros_hop = jnp.zeros((rps, d), jnp.float32)
    zeros_deg = jnp.zeros((rps, DW), jnp.float32)
    ones_deg = jnp.ones((BLK, DW), jnp.float32)

    mesh = plsc.VectorSubcoreMesh(core_axis_name="c", subcore_axis_name="s")

    deg_call = pl.kernel(
        functools.partial(_deg_kernel_body, n_pad, bpw),
        out_type=jax.ShapeDtypeStruct((NC, n_pad, DW), jnp.float32),
        mesh=mesh,
        scratch_types=[
            pltpu.VMEM((bpw, BLK), jnp.int32),
            pltpu.VMEM((BLK, DW), jnp.float32),
            pltpu.VMEM_SHARED((n_pad, DW), jnp.float32),
        ],
    )

    hop_call = pl.kernel(
        functools.partial(_hop_kernel_body, n_pad, d, bpw),
        out_type=jax.ShapeDtypeStruct((NC, n_pad, d), jnp.float32),
        mesh=mesh,
        scratch_types=[
            pltpu.VMEM((bpw, BLK), jnp.int32),
            pltpu.VMEM((bpw, BLK), jnp.int32),
            pltpu.VMEM((BLK, d), jnp.float32),
            pltpu.VMEM_SHARED((n_pad, d), jnp.float32),
            pltpu.SemaphoreType.DMA,
        ],
    )

    prep_call = pl.pallas_call(
        _prep_body,
        out_shape=[
            jax.ShapeDtypeStruct((n_pad, d), jnp.float32),
            jax.ShapeDtypeStruct((n_pad, 1), jnp.float32),
            jax.ShapeDtypeStruct((n_pad, 1), jnp.float32),
        ],
    )

    comb_call = pl.pallas_call(
        _comb_body,
        out_shape=jax.ShapeDtypeStruct((n_pad, d), jnp.float32),
    )

    final_call = pl.pallas_call(
        functools.partial(_final_body, n),
        out_shape=jax.ShapeDtypeStruct((n, d), jnp.float32),
    )

    parts = deg_call(dst2d, zeros_deg, ones_deg)
    g, dinv, dinvv = prep_call(parts, x_pad)
    for k in range(3):
        p = hop_call(g, src2d, dst2d, zeros_hop)
        if k < 2:
            g = comb_call(p, dinvv)
    return final_call(p, dinv, W, b.reshape(1, d))


# speedup vs baseline: 3.1331x; 1.0119x over previous
import functools

import jax
import jax.numpy as jnp
from jax import lax
from jax.experimental import pallas as pl
from jax.experimental.pallas import tpu as pltpu
from jax.experimental.pallas import tpu_sc as plsc

NC = 2
NS = 16
BLK = 128
DW = 16
DF = 128
CH = 8


def _deg_kernel_body(n_pad, bpw, dst_hbm, zeros_hbm, ones_hbm, out_hbm,
                     dstv, onesv, acc):
    c = lax.axis_index("c")
    s = lax.axis_index("s")
    rps = n_pad // NS
    pltpu.sync_copy(zeros_hbm, acc.at[pl.ds(s * rps, rps)])
    pltpu.sync_copy(ones_hbm, onesv)
    plsc.subcore_barrier()

    w = s * NC + c
    pltpu.sync_copy(dst_hbm.at[pl.ds(w * bpw, bpw)], dstv)

    def body(blk, _):
        pltpu.sync_copy(onesv, acc.at[dstv.at[blk]], add=True)
        return 0

    lax.fori_loop(0, bpw, body, 0)
    plsc.subcore_barrier()

    for core_id in range(NC):
        @pl.when(c == core_id)
        def _():
            pltpu.sync_copy(acc.at[pl.ds(s * rps, rps)],
                            out_hbm.at[core_id, pl.ds(s * rps, rps)])


def _hop_kernel_body(n_pad, bpw, g_hbm, src_hbm, dst_hbm,
                     zeros_hbm, out_hbm, srcv, dstv, rows, acc):
    c = lax.axis_index("c")
    s = lax.axis_index("s")
    rps = n_pad // NS
    sl = pl.ds(s * rps, rps)

    pltpu.sync_copy(zeros_hbm, acc.at[sl])
    plsc.subcore_barrier()

    w = s * NC + c

    def chunk(ch, _):
        base = w * bpw + ch * CH
        pltpu.sync_copy(src_hbm.at[pl.ds(base, CH)], srcv)
        pltpu.sync_copy(dst_hbm.at[pl.ds(base, CH)], dstv)

        def body(blk, _):
            pltpu.sync_copy(g_hbm.at[srcv.at[blk]], rows)
            pltpu.sync_copy(rows, acc.at[dstv.at[blk]], add=True)
            return 0

        lax.fori_loop(0, CH, body, 0)
        return 0

    lax.fori_loop(0, bpw // CH, chunk, 0)
    plsc.subcore_barrier()

    for core_id in range(NC):
        @pl.when(c == core_id)
        def _():
            pltpu.sync_copy(acc.at[sl], out_hbm.at[core_id, sl])


def _prep_body(parts_ref, x_ref, g_ref, dinv_ref, dinvv_ref):
    deg = jnp.maximum(parts_ref[0, :, 0:1] + parts_ref[1, :, 0:1], 1.0)
    dinv = lax.rsqrt(deg)
    g_ref[...] = x_ref[...] * dinv
    dinv_ref[...] = dinv
    dinvv_ref[...] = 1.0 / deg


def _scale_body(p_ref, s_ref, g_ref):
    g_ref[...] = (p_ref[0] + p_ref[1]) * s_ref[...]


def _final_body(n, p_ref, dinv_ref, w_ref, b_ref, o_ref):
    h = (p_ref[0, :n] + p_ref[1, :n]) * dinv_ref[:n]
    o_ref[...] = lax.dot_general(
        h, w_ref[...], (((1,), (1,)), ((), ())),
        preferred_element_type=jnp.float32,
        precision=lax.Precision.HIGHEST) + b_ref[...]


def kernel(x, edge_index, W, b):
    n, d = x.shape
    e = edge_index.shape[1]
    n_pad = ((n + 1279) // 1280) * 1280
    e_tot = e + n
    quant = NC * NS * BLK * 8
    e_pad = ((e_tot + quant - 1) // quant) * quant
    bpw = e_pad // (NC * NS * BLK)
    rps = n_pad // NS

    src = edge_index[0]
    dst = edge_index[1]
    loop = jnp.arange(n, dtype=jnp.int32)
    padv = jnp.full((e_pad - e_tot,), n, dtype=jnp.int32)
    src2d = jnp.concatenate([src, loop, padv]).reshape(e_pad // BLK, BLK)
    dst2d = jnp.concatenate([dst, loop, padv]).reshape(e_pad // BLK, BLK)
    x_pad = jnp.pad(x, ((0, n_pad - n), (0, 0)))
    zeros_hop = jnp.zeros((rps, DF), jnp.float32)
    zeros_deg = jnp.zeros((rps, DW), jnp.float32)
    ones_deg = jnp.ones((BLK, DW), jnp.float32)

    mesh = plsc.VectorSubcoreMesh(core_axis_name="c", subcore_axis_name="s")

    deg_call = pl.kernel(
        functools.partial(_deg_kernel_body, n_pad, bpw),
        out_type=jax.ShapeDtypeStruct((NC, n_pad, DW), jnp.float32),
        mesh=mesh,
        scratch_types=[
            pltpu.VMEM((bpw, BLK), jnp.int32),
            pltpu.VMEM((BLK, DW), jnp.float32),
            pltpu.VMEM_SHARED((n_pad, DW), jnp.float32),
        ],
    )

    hop_call = pl.kernel(
        functools.partial(_hop_kernel_body, n_pad, bpw),
        out_type=jax.ShapeDtypeStruct((NC, n_pad, DF), jnp.float32),
        mesh=mesh,
        scratch_types=[
            pltpu.VMEM((CH, BLK), jnp.int32),
            pltpu.VMEM((CH, BLK), jnp.int32),
            pltpu.VMEM((BLK, DF), jnp.float32),
            pltpu.VMEM_SHARED((n_pad, DF), jnp.float32),
        ],
    )

    prep_call = pl.pallas_call(
        _prep_body,
        out_shape=[
            jax.ShapeDtypeStruct((n_pad, DF), jnp.float32),
            jax.ShapeDtypeStruct((n_pad, 1), jnp.float32),
            jax.ShapeDtypeStruct((n_pad, 1), jnp.float32),
        ],
    )

    scale_call = pl.pallas_call(
        _scale_body,
        out_shape=jax.ShapeDtypeStruct((n_pad, DF), jnp.float32),
    )

    final_call = pl.pallas_call(
        functools.partial(_final_body, n),
        out_shape=jax.ShapeDtypeStruct((n, d), jnp.float32),
    )

    parts = deg_call(dst2d, zeros_deg, ones_deg)
    g, dinv, dinvv = prep_call(parts, x_pad)
    for k in range(3):
        p = hop_call(g, src2d, dst2d, zeros_hop)
        if k < 2:
            g = scale_call(p, dinvv)
    return final_call(p, dinv, W, b.reshape(1, d))
